# baseline (speedup 1.0000x reference)
import dataclasses
import functools

import jax
import jax.numpy as jnp
from jax import lax
from jax.experimental import pallas as pl
from jax.experimental.pallas import tpu as pltpu
from jax.experimental.pallas import tpu_sc as plsc

B = 4096
N = 200
C = 64
R = N * B

NC = 2
NS = 16
NW = NC * NS
ROWS_PER_W = R // NW
CHUNK = 128
CHUNKS_PER_W = ROWS_PER_W // CHUNK
BBLKS = B // CHUNK


def _sc_compiler_params():
    cp = pltpu.CompilerParams(use_tc_tiling_on_sc=False)
    if "needs_layout_passes" in pltpu.CompilerParams.__dataclass_fields__:
        cp = dataclasses.replace(cp, needs_layout_passes=False)
    return cp


def _sc_gather_sum(idx_s, idx_c, shape_table, color_table):
    mesh = plsc.VectorSubcoreMesh(core_axis_name="c", subcore_axis_name="s")

    T = CHUNKS_PER_W

    @functools.partial(
        pl.kernel,
        out_type=jax.ShapeDtypeStruct((N * 4, BBLKS, 8, CHUNK), jnp.int32),
        mesh=mesh,
        scratch_types=[
            pltpu.VMEM((4, CHUNK), jnp.int32),
            pltpu.VMEM((4, CHUNK), jnp.int32),
            pltpu.VMEM((4, CHUNK, C // 2), jnp.int32),
            pltpu.VMEM((1000 * (C // 2),), jnp.int32),
            pltpu.VMEM((2, 4, 8, CHUNK + 1), jnp.int32),
        ] + [pltpu.SemaphoreType.DMA] * 10,
        compiler_params=_sc_compiler_params(),
    )
    def k(idx_s_hbm, idx_c_hbm, stab_hbm, ctab_hbm, out_hbm,
          idxs_v, idxc_v, rows_s, ctab_v, out_v,
          is0, is1, is2, is3, gs0, gs1, gs2, gs3, ws0, ws1):
        isem = (is0, is1, is2, is3)
        gsem = (gs0, gs1, gs2, gs3)
        wsem = (ws0, ws1)
        wid = lax.axis_index("s") * NC + lax.axis_index("c")
        base = wid * ROWS_PER_W
        cbase = wid * T

        def fire_idx(t, s):
            pltpu.async_copy(idx_s_hbm.at[pl.ds(base + t * CHUNK, CHUNK)],
                             idxs_v.at[s], isem[s])
            pltpu.async_copy(idx_c_hbm.at[pl.ds(base + t * CHUNK, CHUNK)],
                             idxc_v.at[s], isem[s])

        def drain_idx(s):
            pltpu.make_async_copy(idx_s_hbm.at[pl.ds(0, CHUNK)],
                                  idxs_v.at[s], isem[s]).wait()
            pltpu.make_async_copy(idx_c_hbm.at[pl.ds(0, CHUNK)],
                                  idxc_v.at[s], isem[s]).wait()

        def fire_gathers(s):
            pltpu.async_copy(stab_hbm.at[idxs_v.at[s]], rows_s.at[s], gsem[s])

        def drain_gather(s):
            pltpu.make_async_copy(stab_hbm.at[pl.ds(0, CHUNK)],
                                  rows_s.at[s], gsem[s]).wait()

        def out_src(w):
            return out_v.at[w, :, :, pl.ds(0, CHUNK)]

        def drain_write(w):
            pltpu.make_async_copy(out_hbm.at[pl.ds(0, 4), 0],
                                  out_src(w), wsem[w]).wait()

        iota = lax.iota(jnp.int32, 16)

        pltpu.sync_copy(ctab_hbm, ctab_v)

        for s in range(4):
            fire_idx(s, s)
        for q in (0, 1):
            drain_idx(q)
            fire_gathers(q)

        @pl.loop(0, T // 4)
        def _(g):
            for p in range(4):
                t = g * 4 + p
                gt = cbase + t
                n4 = (gt >> 5) * 4
                bb = gt & (BBLKS - 1)

                @pl.when(t + 2 < T)
                def _():
                    drain_idx((p + 2) % 4)
                    fire_gathers((p + 2) % 4)

                drain_gather(p)

                @pl.when(t >= 2)
                def _():
                    drain_write(p % 2)

                @pl.loop(0, CHUNK // 16)
                def _(i0):
                    civ = idxc_v[p, pl.ds(i0 * 16, 16)] * (C // 2)
                    for q in range(16):
                        i = i0 * 16 + q
                        blv = jnp.full((16,), i, jnp.int32)
                        cb = civ[q]
                        for j in range(C // 32):
                            cv = iota + (16 * j)
                            xs = plsc.bitcast(rows_s[p, i, pl.ds(16 * j, 16)],
                                              jnp.bfloat16)
                            xc = plsc.bitcast(
                                plsc.load_gather(ctab_v, [cv + cb]),
                                jnp.bfloat16)
                            w = plsc.bitcast(xs + xc, jnp.int32)
                            plsc.store_scatter(
                                out_v.at[p % 2],
                                [lax.shift_right_logical(cv, 3),
                                 lax.bitwise_and(cv, 7), blv], w)

                @pl.when(t + 4 < T)
                def _():
                    fire_idx(t + 4, p)

                pltpu.async_copy(out_src(p % 2),
                                 out_hbm.at[pl.ds(n4, 4), bb], wsem[p % 2])

        drain_write(0)
        drain_write(1)

    return k(idx_s, idx_c, shape_table, color_table)


def _tc_pose_add(pose_t, sum4d, Wt, b2d):
    NBLK = 4

    def body(pose_ref, sum_ref, wt_ref, b_ref, out_ref):
        for nn in range(NBLK):
            mm = jnp.dot(wt_ref[...], pose_ref[nn],
                         preferred_element_type=jnp.float32) + b_ref[...]
            for t in range(4):
                sw = jnp.transpose(sum_ref[nn * 4 + t],
                                   (1, 0, 2)).reshape(8, B)
                f_lo = lax.bitcast_convert_type(
                    lax.shift_left(sw, 16), jnp.float32)
                f_hi = lax.bitcast_convert_type(
                    lax.bitwise_and(sw, jnp.int32(-65536)), jnp.float32)
                out_ref[nn, pl.ds(16 * t, 8), :] = (
                    mm[16 * t:16 * t + 8, :] + f_lo)
                out_ref[nn, pl.ds(16 * t + 8, 8), :] = (
                    mm[16 * t + 8:16 * t + 16, :] + f_hi)

    return pl.pallas_call(
        body,
        grid=(N // NBLK,),
        in_specs=[
            pl.BlockSpec((NBLK, 16, B), lambda i: (i, 0, 0)),
            pl.BlockSpec((NBLK * 4, BBLKS, 8, CHUNK), lambda i: (i, 0, 0, 0)),
            pl.BlockSpec((C, 16), lambda i: (0, 0)),
            pl.BlockSpec((C, 1), lambda i: (0, 0)),
        ],
        out_specs=pl.BlockSpec((NBLK, C, B), lambda i: (i, 0, 0)),
        out_shape=jax.ShapeDtypeStruct((N, C, B), jnp.float32),
    )(pose_t, sum4d, Wt, b2d)


def _pack_table(tab):
    v = tab.shape[0]
    u = lax.bitcast_convert_type(tab, jnp.uint32)
    r = (u + jnp.uint32(0x7FFF) + ((u >> 16) & 1)) >> 16
    r4 = r.reshape(v, C // 16, 2, 8)
    w = r4[:, :, 0, :] | (r4[:, :, 1, :] << 16)
    return lax.bitcast_convert_type(w.reshape(v, C // 2), jnp.int32)


def kernel(shape, color, pose, shape_table, color_table, W, b):
    idx_s = shape.astype(jnp.int32).T.reshape(R)
    idx_c = color.astype(jnp.int32).T.reshape(R)
    sum4d = _sc_gather_sum(idx_s, idx_c, _pack_table(shape_table),
                           _pack_table(color_table).reshape(-1))

    pose_t = pose.transpose(1, 2, 0)
    out_cb = _tc_pose_add(pose_t, sum4d, W.T, b.reshape(C, 1))
    return out_cb.transpose(0, 2, 1)


# speedup vs baseline: 1.1901x; 1.1901x over previous
import dataclasses
import functools

import jax
import jax.numpy as jnp
from jax import lax
from jax.experimental import pallas as pl
from jax.experimental.pallas import tpu as pltpu
from jax.experimental.pallas import tpu_sc as plsc

B = 4096
N = 200
C = 64
R = N * B

NC = 2
NS = 16
NW = NC * NS
ROWS_PER_W = R // NW
CHUNK = 128
CHUNKS_PER_W = ROWS_PER_W // CHUNK
BBLKS = B // CHUNK


def _sc_compiler_params():
    cp = pltpu.CompilerParams(use_tc_tiling_on_sc=False)
    if "needs_layout_passes" in pltpu.CompilerParams.__dataclass_fields__:
        cp = dataclasses.replace(cp, needs_layout_passes=False)
    return cp


SLICES = 2
NSL = N // SLICES
RSL = NSL * B


def _sc_gather_sum(idx_s, idx_c, shape_table, color_table):
    mesh = plsc.VectorSubcoreMesh(core_axis_name="c", subcore_axis_name="s")

    T = RSL // NW // CHUNK
    rows_w = RSL // NW

    @functools.partial(
        pl.kernel,
        out_type=jax.ShapeDtypeStruct((NSL * 4, BBLKS, 8, CHUNK), jnp.int32),
        mesh=mesh,
        scratch_types=[
            pltpu.VMEM((4, CHUNK), jnp.int32),
            pltpu.VMEM((4, CHUNK), jnp.int32),
            pltpu.VMEM((4, CHUNK, C // 2), jnp.int32),
            pltpu.VMEM((4, CHUNK, C // 2), jnp.int32),
            pltpu.VMEM((2, 4, 8, CHUNK + 1), jnp.int32),
        ] + [pltpu.SemaphoreType.DMA] * 10,
        compiler_params=_sc_compiler_params(),
    )
    def k(idx_s_hbm, idx_c_hbm, stab_hbm, ctab_hbm, out_hbm,
          idxs_v, idxc_v, rows_s, rows_c, out_v,
          is0, is1, is2, is3, gs0, gs1, gs2, gs3, ws0, ws1):
        isem = (is0, is1, is2, is3)
        gsem = (gs0, gs1, gs2, gs3)
        wsem = (ws0, ws1)
        wid = lax.axis_index("s") * NC + lax.axis_index("c")
        base = wid * rows_w
        cbase = wid * T

        def fire_idx(t, s):
            pltpu.async_copy(idx_s_hbm.at[pl.ds(base + t * CHUNK, CHUNK)],
                             idxs_v.at[s], isem[s])
            pltpu.async_copy(idx_c_hbm.at[pl.ds(base + t * CHUNK, CHUNK)],
                             idxc_v.at[s], isem[s])

        def drain_idx(s):
            pltpu.make_async_copy(idx_s_hbm.at[pl.ds(0, CHUNK)],
                                  idxs_v.at[s], isem[s]).wait()
            pltpu.make_async_copy(idx_c_hbm.at[pl.ds(0, CHUNK)],
                                  idxc_v.at[s], isem[s]).wait()

        def fire_gathers(s):
            pltpu.async_copy(stab_hbm.at[idxs_v.at[s]], rows_s.at[s], gsem[s])
            pltpu.async_copy(ctab_hbm.at[idxc_v.at[s]], rows_c.at[s], gsem[s])

        def drain_gather(s):
            pltpu.make_async_copy(stab_hbm.at[pl.ds(0, CHUNK)],
                                  rows_s.at[s], gsem[s]).wait()
            pltpu.make_async_copy(ctab_hbm.at[pl.ds(0, CHUNK)],
                                  rows_c.at[s], gsem[s]).wait()

        def out_src(w):
            return out_v.at[w, :, :, pl.ds(0, CHUNK)]

        def drain_write(w):
            pltpu.make_async_copy(out_hbm.at[pl.ds(0, 4), 0],
                                  out_src(w), wsem[w]).wait()

        iota = lax.iota(jnp.int32, 16)

        for s in range(4):
            fire_idx(s, s)
        for q in (0, 1):
            drain_idx(q)
            fire_gathers(q)

        @pl.loop(0, T // 4)
        def _(g):
            for p in range(4):
                t = g * 4 + p
                gt = cbase + t
                n4 = (gt >> 5) * 4
                bb = gt & (BBLKS - 1)

                @pl.when(t + 2 < T)
                def _():
                    drain_idx((p + 2) % 4)
                    fire_gathers((p + 2) % 4)

                drain_gather(p)

                @pl.when(t + 4 < T)
                def _():
                    fire_idx(t + 4, p)

                @pl.when(t >= 2)
                def _():
                    drain_write(p % 2)

                @pl.loop(0, CHUNK // 8)
                def _(i0):
                    for q in range(8):
                        i = i0 * 8 + q
                        blv = jnp.full((16,), i, jnp.int32)
                        for j in range(C // 32):
                            cv = iota + (16 * j)
                            xs = plsc.bitcast(rows_s[p, i, pl.ds(16 * j, 16)],
                                              jnp.bfloat16)
                            xc = plsc.bitcast(rows_c[p, i, pl.ds(16 * j, 16)],
                                              jnp.bfloat16)
                            w = plsc.bitcast(xs + xc, jnp.int32)
                            plsc.store_scatter(
                                out_v.at[p % 2],
                                [lax.shift_right_logical(cv, 3),
                                 lax.bitwise_and(cv, 7), blv], w)

                pltpu.async_copy(out_src(p % 2),
                                 out_hbm.at[pl.ds(n4, 4), bb], wsem[p % 2])

        drain_write(0)
        drain_write(1)

    return k(idx_s, idx_c, shape_table, color_table)


def _tc_pose_add(pose_t, sum4d, Wt, b2d, sl, prev):
    NBLK = 4
    off = sl * (NSL // NBLK)

    def body(pose_ref, sum_ref, wt_ref, b_ref, *refs):
        out_ref = refs[-1]
        for nn in range(NBLK):
            mm = jnp.dot(wt_ref[...], pose_ref[nn],
                         preferred_element_type=jnp.float32) + b_ref[...]
            for t in range(4):
                sw = jnp.transpose(sum_ref[nn * 4 + t],
                                   (1, 0, 2)).reshape(8, B)
                f_lo = lax.bitcast_convert_type(
                    lax.shift_left(sw, 16), jnp.float32)
                f_hi = lax.bitcast_convert_type(
                    lax.bitwise_and(sw, jnp.int32(-65536)), jnp.float32)
                out_ref[nn, pl.ds(16 * t, 8), :] = (
                    mm[16 * t:16 * t + 8, :] + f_lo)
                out_ref[nn, pl.ds(16 * t + 8, 8), :] = (
                    mm[16 * t + 8:16 * t + 16, :] + f_hi)

    in_specs = [
        pl.BlockSpec((NBLK, 16, B), lambda i: (i + off, 0, 0)),
        pl.BlockSpec((NBLK * 4, BBLKS, 8, CHUNK), lambda i: (i, 0, 0, 0)),
        pl.BlockSpec((C, 16), lambda i: (0, 0)),
        pl.BlockSpec((C, 1), lambda i: (0, 0)),
    ]
    args = [pose_t, sum4d, Wt, b2d]
    aliases = {}
    if prev is not None:
        in_specs.append(pl.BlockSpec(memory_space=pl.ANY))
        args.append(prev)
        aliases = {4: 0}
    return pl.pallas_call(
        body,
        grid=(NSL // NBLK,),
        in_specs=in_specs,
        out_specs=pl.BlockSpec((NBLK, C, B), lambda i: (i + off, 0, 0)),
        out_shape=jax.ShapeDtypeStruct((N, C, B), jnp.float32),
        input_output_aliases=aliases,
    )(*args)


def _pack_table(tab):
    v = tab.shape[0]
    u = lax.bitcast_convert_type(tab, jnp.uint32)
    r = (u + jnp.uint32(0x7FFF) + ((u >> 16) & 1)) >> 16
    r4 = r.reshape(v, C // 16, 2, 8)
    w = r4[:, :, 0, :] | (r4[:, :, 1, :] << 16)
    return lax.bitcast_convert_type(w.reshape(v, C // 2), jnp.int32)


def kernel(shape, color, pose, shape_table, color_table, W, b):
    idx_s = shape.astype(jnp.int32).T.reshape(R)
    idx_c = color.astype(jnp.int32).T.reshape(R)
    stab = _pack_table(shape_table)
    ctab = _pack_table(color_table)
    pose_t = pose.transpose(1, 2, 0)
    Wt = W.T
    b2d = b.reshape(C, 1)

    out = None
    for sl in range(SLICES):
        r0 = sl * RSL
        sum4d = _sc_gather_sum(
            lax.dynamic_slice_in_dim(idx_s, r0, RSL),
            lax.dynamic_slice_in_dim(idx_c, r0, RSL), stab, ctab)
        out = _tc_pose_add(pose_t, sum4d, Wt, b2d, sl, out)
    return out.transpose(0, 2, 1)


# speedup vs baseline: 1.2192x; 1.0245x over previous
import dataclasses
import functools

import jax
import jax.numpy as jnp
from jax import lax
from jax.experimental import pallas as pl
from jax.experimental.pallas import tpu as pltpu
from jax.experimental.pallas import tpu_sc as plsc

B = 4096
N = 200
C = 64
R = N * B

NC = 2
NS = 16
NW = NC * NS
ROWS_PER_W = R // NW
CHUNK = 128
CHUNKS_PER_W = ROWS_PER_W // CHUNK
BBLKS = B // CHUNK


def _sc_compiler_params():
    cp = pltpu.CompilerParams(use_tc_tiling_on_sc=False)
    if "needs_layout_passes" in pltpu.CompilerParams.__dataclass_fields__:
        cp = dataclasses.replace(cp, needs_layout_passes=False)
    return cp


SLICES = 4
NSL = N // SLICES
RSL = NSL * B


def _sc_gather_sum(idx_s, idx_c, shape_table, color_table):
    mesh = plsc.VectorSubcoreMesh(core_axis_name="c", subcore_axis_name="s")

    T = RSL // NW // CHUNK
    rows_w = RSL // NW

    @functools.partial(
        pl.kernel,
        out_type=jax.ShapeDtypeStruct((NSL * 4, BBLKS, 8, CHUNK), jnp.int32),
        mesh=mesh,
        scratch_types=[
            pltpu.VMEM((4, CHUNK), jnp.int32),
            pltpu.VMEM((4, CHUNK), jnp.int32),
            pltpu.VMEM((4, CHUNK, C // 2), jnp.int32),
            pltpu.VMEM((4, CHUNK, C // 2), jnp.int32),
            pltpu.VMEM((2, 4, 8, CHUNK + 1), jnp.int32),
        ] + [pltpu.SemaphoreType.DMA] * 10,
        compiler_params=_sc_compiler_params(),
    )
    def k(idx_s_hbm, idx_c_hbm, stab_hbm, ctab_hbm, out_hbm,
          idxs_v, idxc_v, rows_s, rows_c, out_v,
          is0, is1, is2, is3, gs0, gs1, gs2, gs3, ws0, ws1):
        isem = (is0, is1, is2, is3)
        gsem = (gs0, gs1, gs2, gs3)
        wsem = (ws0, ws1)
        wid = lax.axis_index("s") * NC + lax.axis_index("c")
        base = wid * rows_w
        cbase = wid * T

        def fire_idx(t, s):
            pltpu.async_copy(idx_s_hbm.at[pl.ds(base + t * CHUNK, CHUNK)],
                             idxs_v.at[s], isem[s])
            pltpu.async_copy(idx_c_hbm.at[pl.ds(base + t * CHUNK, CHUNK)],
                             idxc_v.at[s], isem[s])

        def drain_idx(s):
            pltpu.make_async_copy(idx_s_hbm.at[pl.ds(0, CHUNK)],
                                  idxs_v.at[s], isem[s]).wait()
            pltpu.make_async_copy(idx_c_hbm.at[pl.ds(0, CHUNK)],
                                  idxc_v.at[s], isem[s]).wait()

        def fire_gathers(s):
            pltpu.async_copy(stab_hbm.at[idxs_v.at[s]], rows_s.at[s], gsem[s])
            pltpu.async_copy(ctab_hbm.at[idxc_v.at[s]], rows_c.at[s], gsem[s])

        def drain_gather(s):
            pltpu.make_async_copy(stab_hbm.at[pl.ds(0, CHUNK)],
                                  rows_s.at[s], gsem[s]).wait()
            pltpu.make_async_copy(ctab_hbm.at[pl.ds(0, CHUNK)],
                                  rows_c.at[s], gsem[s]).wait()

        def out_src(w):
            return out_v.at[w, :, :, pl.ds(0, CHUNK)]

        def drain_write(w):
            pltpu.make_async_copy(out_hbm.at[pl.ds(0, 4), 0],
                                  out_src(w), wsem[w]).wait()

        iota = lax.iota(jnp.int32, 16)

        for s in range(4):
            fire_idx(s, s)
        for q in (0, 1):
            drain_idx(q)
            fire_gathers(q)

        @pl.loop(0, T // 4)
        def _(g):
            for p in range(4):
                t = g * 4 + p
                gt = cbase + t
                n4 = (gt >> 5) * 4
                bb = gt & (BBLKS - 1)

                @pl.when(t + 2 < T)
                def _():
                    drain_idx((p + 2) % 4)
                    fire_gathers((p + 2) % 4)

                drain_gather(p)

                @pl.when(t + 4 < T)
                def _():
                    fire_idx(t + 4, p)

                @pl.when(t >= 2)
                def _():
                    drain_write(p % 2)

                @pl.loop(0, CHUNK // 8)
                def _(i0):
                    for q in range(8):
                        i = i0 * 8 + q
                        blv = jnp.full((16,), i, jnp.int32)
                        for j in range(C // 32):
                            cv = iota + (16 * j)
                            xs = plsc.bitcast(rows_s[p, i, pl.ds(16 * j, 16)],
                                              jnp.bfloat16)
                            xc = plsc.bitcast(rows_c[p, i, pl.ds(16 * j, 16)],
                                              jnp.bfloat16)
                            w = plsc.bitcast(xs + xc, jnp.int32)
                            plsc.store_scatter(
                                out_v.at[p % 2],
                                [lax.shift_right_logical(cv, 3),
                                 lax.bitwise_and(cv, 7), blv], w)

                pltpu.async_copy(out_src(p % 2),
                                 out_hbm.at[pl.ds(n4, 4), bb], wsem[p % 2])

        drain_write(0)
        drain_write(1)

    return k(idx_s, idx_c, shape_table, color_table)


def _tc_pose_add(pose_t, sum4d, Wt, b2d, sl, prev):
    NBLK = 4
    off = sl * (NSL // NBLK)

    def body(pose_ref, sum_ref, wt_ref, b_ref, *refs):
        out_ref = refs[-1]
        for nn in range(NBLK):
            mm = jnp.dot(wt_ref[...], pose_ref[nn],
                         preferred_element_type=jnp.float32) + b_ref[...]
            for t in range(4):
                sw = jnp.transpose(sum_ref[nn * 4 + t],
                                   (1, 0, 2)).reshape(8, B)
                f_lo = lax.bitcast_convert_type(
                    lax.shift_left(sw, 16), jnp.float32)
                f_hi = lax.bitcast_convert_type(
                    lax.bitwise_and(sw, jnp.int32(-65536)), jnp.float32)
                out_ref[nn, pl.ds(16 * t, 8), :] = (
                    mm[16 * t:16 * t + 8, :] + f_lo)
                out_ref[nn, pl.ds(16 * t + 8, 8), :] = (
                    mm[16 * t + 8:16 * t + 16, :] + f_hi)

    in_specs = [
        pl.BlockSpec((NBLK, 16, B), lambda i: (i + off, 0, 0)),
        pl.BlockSpec((NBLK * 4, BBLKS, 8, CHUNK), lambda i: (i, 0, 0, 0)),
        pl.BlockSpec((C, 16), lambda i: (0, 0)),
        pl.BlockSpec((C, 1), lambda i: (0, 0)),
    ]
    args = [pose_t, sum4d, Wt, b2d]
    aliases = {}
    if prev is not None:
        in_specs.append(pl.BlockSpec(memory_space=pl.ANY))
        args.append(prev)
        aliases = {4: 0}
    return pl.pallas_call(
        body,
        grid=(NSL // NBLK,),
        in_specs=in_specs,
        out_specs=pl.BlockSpec((NBLK, C, B), lambda i: (i + off, 0, 0)),
        out_shape=jax.ShapeDtypeStruct((N, C, B), jnp.float32),
        input_output_aliases=aliases,
    )(*args)


def _pack_table(tab):
    v = tab.shape[0]
    u = lax.bitcast_convert_type(tab, jnp.uint32)
    r = (u + jnp.uint32(0x7FFF) + ((u >> 16) & 1)) >> 16
    r4 = r.reshape(v, C // 16, 2, 8)
    w = r4[:, :, 0, :] | (r4[:, :, 1, :] << 16)
    return lax.bitcast_convert_type(w.reshape(v, C // 2), jnp.int32)


def kernel(shape, color, pose, shape_table, color_table, W, b):
    idx_s = shape.astype(jnp.int32).T.reshape(R)
    idx_c = color.astype(jnp.int32).T.reshape(R)
    stab = _pack_table(shape_table)
    ctab = _pack_table(color_table)
    pose_t = pose.transpose(1, 2, 0)
    Wt = W.T
    b2d = b.reshape(C, 1)

    out = None
    for sl in range(SLICES):
        r0 = sl * RSL
        sum4d = _sc_gather_sum(
            lax.dynamic_slice_in_dim(idx_s, r0, RSL),
            lax.dynamic_slice_in_dim(idx_c, r0, RSL), stab, ctab)
        out = _tc_pose_add(pose_t, sum4d, Wt, b2d, sl, out)
    return out.transpose(0, 2, 1)
